# baseline (speedup 1.0000x reference)
import functools

import jax
import jax.numpy as jnp
from jax import lax
from jax.experimental import pallas as pl
from jax.experimental.pallas import tpu as pltpu
from jax.experimental.pallas import tpu_sc as plsc

B = 4
T = 4096
D = 768
H = 12
DH = 64
FF = 2816
KCAP = 512
N = B * KCAP
BT = B * T
EPS = 1e-6

f32 = jnp.float32
i32 = jnp.int32
bf16 = jnp.bfloat16


def _scores_body(beta_ref, o_ref, p_ref, pr_ref, s_ref):
    o = o_ref[...]
    p = p_ref[...]
    pr = pr_ref[...]
    d1 = o - p
    d2 = p - pr
    cu = jnp.sqrt(jnp.sum(d1 * d1, axis=1, keepdims=True))
    ce = jnp.sqrt(jnp.sum(d2 * d2, axis=1, keepdims=True))
    s_ref[...] = beta_ref[0, 0] * cu + beta_ref[0, 1] * (ce + beta_ref[0, 2])


def _topk_body(scol_ref, srow_ref, idx_ref, idx2_ref, par_ref, gate_ref):
    for b in range(B):
        scol = scol_ref[b * T:(b + 1) * T, :]
        ii = lax.broadcasted_iota(i32, (T, KCAP), 0)
        rank = jnp.zeros((T, 1), f32)
        for c in range(T // KCAP):
            srow_c = srow_ref[:, b * T + c * KCAP: b * T + (c + 1) * KCAP]
            jj = c * KCAP + lax.broadcasted_iota(i32, (T, KCAP), 1)
            gt = srow_c > scol
            tie = (srow_c == scol) & (jj < ii)
            rank = rank + jnp.sum((gt | tie).astype(f32), axis=1, keepdims=True)
        p_iota = lax.broadcasted_iota(i32, (T, KCAP), 1).astype(f32)
        oh = rank == p_iota
        fli = (b * T + lax.broadcasted_iota(i32, (T, KCAP), 0)).astype(f32)
        idx_row = jnp.sum(jnp.where(oh, fli, 0.0), axis=0, keepdims=True)
        sc_row = jnp.sum(jnp.where(oh, scol, 0.0), axis=0, keepdims=True)
        idx_i = idx_row.astype(i32)
        idx_ref[b:b + 1, :] = idx_i
        idx2_ref[b:b + 1, :] = idx_i >> 1
        par_ref[b:b + 1, :] = (idx_i & 1).astype(f32)
        gate_ref[b:b + 1, :] = 1.0 / (1.0 + jnp.exp(-sc_row))


_NW = 32
_GW = N // _NW
_SLAB = BT // _NW
_CCH = 64


@functools.cache
def _sc_kernels():
    vmesh = plsc.VectorSubcoreMesh(core_axis_name="c", subcore_axis_name="s")

    @functools.partial(
        pl.kernel,
        out_type=(
            jax.ShapeDtypeStruct((N, D), f32),
            jax.ShapeDtypeStruct((N, 2 * DH), f32),
            jax.ShapeDtypeStruct((N, 2 * DH), f32),
        ),
        mesh=vmesh,
        scratch_types=[
            pltpu.VMEM((_GW,), i32),
            pltpu.VMEM((_GW,), i32),
            pltpu.VMEM((_GW, D), f32),
            pltpu.VMEM((_GW, 2 * DH), f32),
            pltpu.VMEM((_GW, 2 * DH), f32),
            pltpu.SemaphoreType.DMA,
        ],
    )
    def sc_gather(hid_hbm, cp_hbm, sp_hbm, idx_hbm, idx2_hbm,
                  sel_hbm, cpg_hbm, spg_hbm,
                  idx_v, idx2_v, rows_v, c_v, s_v, sem):
        wid = lax.axis_index("s") * 2 + lax.axis_index("c")
        base = wid * _GW
        pltpu.sync_copy(idx_hbm.at[pl.ds(base, _GW)], idx_v)
        pltpu.sync_copy(idx2_hbm.at[pl.ds(base, _GW)], idx2_v)
        pltpu.async_copy(hid_hbm.at[idx_v], rows_v, sem).wait()
        pltpu.async_copy(cp_hbm.at[idx2_v], c_v, sem).wait()
        pltpu.async_copy(sp_hbm.at[idx2_v], s_v, sem).wait()
        pltpu.sync_copy(rows_v, sel_hbm.at[pl.ds(base, _GW)])
        pltpu.sync_copy(c_v, cpg_hbm.at[pl.ds(base, _GW)])
        pltpu.sync_copy(s_v, spg_hbm.at[pl.ds(base, _GW)])

    @functools.partial(
        pl.kernel,
        out_type=jax.ShapeDtypeStruct((BT, D), f32),
        mesh=vmesh,
        scratch_types=[
            pltpu.VMEM((_GW,), i32),
            pltpu.VMEM((_CCH, D), f32),
            pltpu.VMEM((_CCH, D), f32),
            pltpu.SemaphoreType.DMA,
            pltpu.SemaphoreType.DMA,
            pltpu.SemaphoreType.DMA,
            pltpu.SemaphoreType.DMA,
        ],
    )
    def sc_scatter(hid_hbm, upd_hbm, idx_hbm, out_hbm,
                   idx_v, buf_a, buf_b, lsem_a, lsem_b, ssem_a, ssem_b):
        c = lax.axis_index("c")
        s = lax.axis_index("s")
        slab = (c * 16 + s) * _SLAB
        bufs = (buf_a, buf_b)
        lsems = (lsem_a, lsem_b)
        ssems = (ssem_a, ssem_b)
        prev_store = [None, None]
        for i in range(_SLAB // _CCH):
            j = i % 2
            if prev_store[j] is not None:
                prev_store[j].wait()
            pltpu.async_copy(hid_hbm.at[pl.ds(slab + i * _CCH, _CCH)],
                             bufs[j], lsems[j]).wait()
            prev_store[j] = pltpu.async_copy(
                bufs[j], out_hbm.at[pl.ds(slab + i * _CCH, _CCH)], ssems[j])
        for j in range(2):
            if prev_store[j] is not None:
                prev_store[j].wait()
        plsc.subcore_barrier()
        pbase = c * (N // 2) + s * _GW
        pltpu.sync_copy(idx_hbm.at[pl.ds(pbase, _GW)], idx_v)
        pltpu.async_copy(upd_hbm.at[pl.ds(pbase, _GW)], buf_a, lsem_a).wait()
        pltpu.sync_copy(buf_a, out_hbm.at[idx_v])

    return sc_gather, sc_scatter


def _rotate_half_full(x):
    left = pltpu.roll(x, x.shape[1] - 32, 1)
    right = pltpu.roll(x, 32, 1)
    lane = lax.broadcasted_iota(i32, x.shape, 1)
    return jnp.where((lane & 32) == 0, -left, right)


def _qkv_body(sel_ref, ln1_ref, wq_ref, bq_ref, wk_ref, bk_ref,
              wv_ref, bv_ref, cp_ref, sp_ref, par_ref,
              q_ref, k_ref, v_ref):
    x = sel_ref[...]
    ms = jnp.mean(x * x, axis=1, keepdims=True)
    hn = (x * lax.rsqrt(ms + EPS) * ln1_ref[...]).astype(bf16)
    q = jnp.dot(hn, wq_ref[...], preferred_element_type=f32) + bq_ref[...]
    k = jnp.dot(hn, wk_ref[...], preferred_element_type=f32) + bk_ref[...]
    v = jnp.dot(hn, wv_ref[...], preferred_element_type=f32) + bv_ref[...]
    par = par_ref[...]
    cp = cp_ref[...]
    sp = sp_ref[...]
    cos = jnp.where(par > 0.5, cp[:, DH:], cp[:, :DH])
    sin = jnp.where(par > 0.5, sp[:, DH:], sp[:, :DH])
    cosf = jnp.concatenate([cos] * H, axis=1)
    sinf = jnp.concatenate([sin] * H, axis=1)
    qr = (q * cosf + _rotate_half_full(q) * sinf) * 0.125
    kr = k * cosf + _rotate_half_full(k) * sinf
    q_ref[...] = qr.astype(bf16)
    k_ref[...] = kr.astype(bf16)
    v_ref[...] = v.astype(bf16)


def _attn_body(q_ref, k_ref, v_ref, o_ref):
    outs = []
    for hh in range(2):
        q1 = q_ref[:, hh * DH:(hh + 1) * DH]
        k1 = k_ref[:, hh * DH:(hh + 1) * DH]
        v1 = v_ref[:, hh * DH:(hh + 1) * DH]
        logits = lax.dot_general(q1, k1, (((1,), (1,)), ((), ())),
                                 preferred_element_type=f32)
        m = jnp.max(logits, axis=1, keepdims=True)
        e = jnp.exp(logits - m).astype(bf16)
        s = jnp.sum(e, axis=1, keepdims=True).astype(f32)
        oraw = jnp.dot(e, v1, preferred_element_type=f32)
        outs.append(oraw / s)
    o_ref[...] = jnp.concatenate(outs, axis=1).astype(bf16)


def _oproj_body(o_ref, wo_ref, sel_ref, ln2_ref, gate_ref, p_ref, hn2_ref):
    sl = sel_ref[...]
    h1 = jnp.dot(o_ref[...], wo_ref[...], preferred_element_type=f32) + sl
    p_ref[...] = sl + (h1 - sl) * gate_ref[...]
    ms = jnp.mean(h1 * h1, axis=1, keepdims=True)
    hn2_ref[...] = (h1 * lax.rsqrt(ms + EPS) * ln2_ref[...]).astype(bf16)


def _mlp_body(hn2_ref, wg_ref, wu_ref, wd_ref, p_ref, gate_ref, out_ref):
    fidx = pl.program_id(1)
    hn2 = hn2_ref[...]
    g = jnp.dot(hn2, wg_ref[...], preferred_element_type=f32)
    u = jnp.dot(hn2, wu_ref[...], preferred_element_type=f32)
    a = (g * (1.0 / (1.0 + jnp.exp(-g))) * u).astype(bf16)
    part = jnp.dot(a, wd_ref[...], preferred_element_type=f32)

    @pl.when(fidx == 0)
    def _():
        out_ref[...] = part

    @pl.when(fidx == 1)
    def _():
        out_ref[...] = p_ref[...] + (out_ref[...] + part) * gate_ref[...]


def kernel(hidden_states, original, posterior, prior, cos, sin,
           beta_ce, beta_cu, ce_offset, ln1_w, ln2_w,
           Wq, bq, Wk, bk, Wv, bv, Wo, Wg, Wu, Wd):
    hid2 = hidden_states.reshape(BT, D)
    cpair = cos.reshape(BT // 2, 2 * DH)
    spair = sin.reshape(BT // 2, 2 * DH)
    betas = jnp.stack([beta_cu, beta_ce, ce_offset]).reshape(1, 3)

    rchunk = 2048
    scores_col = pl.pallas_call(
        _scores_body,
        grid=(BT // rchunk,),
        in_specs=[
            pl.BlockSpec(memory_space=pltpu.SMEM),
            pl.BlockSpec((rchunk, D), lambda i: (i, 0)),
            pl.BlockSpec((rchunk, D), lambda i: (i, 0)),
            pl.BlockSpec((rchunk, D), lambda i: (i, 0)),
        ],
        out_specs=pl.BlockSpec((rchunk, 1), lambda i: (i, 0)),
        out_shape=jax.ShapeDtypeStruct((BT, 1), f32),
    )(betas, original.reshape(BT, D), posterior.reshape(BT, D),
      prior.reshape(BT, D))

    selidx4, selidx24, par4, gates4 = pl.pallas_call(
        _topk_body,
        in_specs=[pl.BlockSpec((BT, 1), lambda: (0, 0)),
                  pl.BlockSpec((1, BT), lambda: (0, 0))],
        out_specs=[pl.BlockSpec((B, KCAP), lambda: (0, 0))] * 4,
        out_shape=(jax.ShapeDtypeStruct((B, KCAP), i32),
                   jax.ShapeDtypeStruct((B, KCAP), i32),
                   jax.ShapeDtypeStruct((B, KCAP), f32),
                   jax.ShapeDtypeStruct((B, KCAP), f32)),
    )(scores_col, scores_col.reshape(1, BT))
    selidx = selidx4.reshape(N)
    selidx2 = selidx24.reshape(N)
    parity = par4.reshape(N, 1)
    gates = gates4.reshape(N, 1)

    sc_gather, sc_scatter = _sc_kernels()
    sel, cpg, spg = sc_gather(hid2, cpair, spair, selidx, selidx2)

    qrows = N // 2
    q2, k2, v2 = pl.pallas_call(
        _qkv_body,
        grid=(2,),
        in_specs=[
            pl.BlockSpec((qrows, D), lambda r: (r, 0)),
            pl.BlockSpec((1, D), lambda r: (0, 0)),
            pl.BlockSpec((D, D), lambda r: (0, 0)),
            pl.BlockSpec((1, D), lambda r: (0, 0)),
            pl.BlockSpec((D, D), lambda r: (0, 0)),
            pl.BlockSpec((1, D), lambda r: (0, 0)),
            pl.BlockSpec((D, D), lambda r: (0, 0)),
            pl.BlockSpec((1, D), lambda r: (0, 0)),
            pl.BlockSpec((qrows, 2 * DH), lambda r: (r, 0)),
            pl.BlockSpec((qrows, 2 * DH), lambda r: (r, 0)),
            pl.BlockSpec((qrows, 1), lambda r: (r, 0)),
        ],
        out_specs=[pl.BlockSpec((qrows, D), lambda r: (r, 0))] * 3,
        out_shape=(jax.ShapeDtypeStruct((N, D), bf16),) * 3,
    )(sel, ln1_w.reshape(1, D), Wq.astype(bf16), bq.reshape(1, D),
      Wk.astype(bf16), bk.reshape(1, D), Wv.astype(bf16), bv.reshape(1, D),
      cpg, spg, parity)

    o2 = pl.pallas_call(
        _attn_body,
        grid=(H // 2,),
        in_specs=[pl.BlockSpec((N, 2 * DH), lambda h: (0, h))] * 3,
        out_specs=pl.BlockSpec((N, 2 * DH), lambda h: (0, h)),
        out_shape=jax.ShapeDtypeStruct((N, D), bf16),
    )(q2, k2, v2)

    orows = N // 2
    pgated, hn2 = pl.pallas_call(
        _oproj_body,
        grid=(2,),
        in_specs=[
            pl.BlockSpec((orows, D), lambda r: (r, 0)),
            pl.BlockSpec((D, D), lambda r: (0, 0)),
            pl.BlockSpec((orows, D), lambda r: (r, 0)),
            pl.BlockSpec((1, D), lambda r: (0, 0)),
            pl.BlockSpec((orows, 1), lambda r: (r, 0)),
        ],
        out_specs=[pl.BlockSpec((orows, D), lambda r: (r, 0))] * 2,
        out_shape=(jax.ShapeDtypeStruct((N, D), f32),
                   jax.ShapeDtypeStruct((N, D), bf16)),
    )(o2, Wo.astype(bf16), sel, ln2_w.reshape(1, D), gates)

    mrows = N // 2
    fchunk = FF // 2
    upd = pl.pallas_call(
        _mlp_body,
        grid=(2, 2),
        in_specs=[
            pl.BlockSpec((mrows, D), lambda r, fc: (r, 0)),
            pl.BlockSpec((D, fchunk), lambda r, fc: (0, fc)),
            pl.BlockSpec((D, fchunk), lambda r, fc: (0, fc)),
            pl.BlockSpec((fchunk, D), lambda r, fc: (fc, 0)),
            pl.BlockSpec((mrows, D), lambda r, fc: (r, 0)),
            pl.BlockSpec((mrows, 1), lambda r, fc: (r, 0)),
        ],
        out_specs=pl.BlockSpec((mrows, D), lambda r, fc: (r, 0)),
        out_shape=jax.ShapeDtypeStruct((N, D), f32),
    )(hn2, Wg.astype(bf16), Wu.astype(bf16), Wd.astype(bf16), pgated, gates)

    out2 = sc_scatter(hid2, upd, selidx)
    return out2.reshape(B, T, D)


# speedup vs baseline: 1.0239x; 1.0239x over previous
import functools

import jax
import jax.numpy as jnp
from jax import lax
from jax.experimental import pallas as pl
from jax.experimental.pallas import tpu as pltpu
from jax.experimental.pallas import tpu_sc as plsc
from jax._src.pallas import mpmd as _mpmd

B = 4
T = 4096
D = 768
H = 12
DH = 64
FF = 2816
KCAP = 512
N = B * KCAP
BT = B * T
EPS = 1e-6

f32 = jnp.float32
i32 = jnp.int32
bf16 = jnp.bfloat16


def _scores_body(beta_ref, o_ref, p_ref, pr_ref, s_ref):
    o = o_ref[...]
    p = p_ref[...]
    pr = pr_ref[...]
    d1 = o - p
    d2 = p - pr
    cu = jnp.sqrt(jnp.sum(d1 * d1, axis=1, keepdims=True))
    ce = jnp.sqrt(jnp.sum(d2 * d2, axis=1, keepdims=True))
    s_ref[...] = beta_ref[0, 0] * cu + beta_ref[0, 1] * (ce + beta_ref[0, 2])


def _topk_body(scol_ref, srow_ref, idx_ref, idx2_ref, par_ref, gate_ref):
    for b in range(B):
        scol = scol_ref[b * T:(b + 1) * T, :]
        ii = lax.broadcasted_iota(i32, (T, KCAP), 0)
        rank = jnp.zeros((T, 1), f32)
        for c in range(T // KCAP):
            srow_c = srow_ref[:, b * T + c * KCAP: b * T + (c + 1) * KCAP]
            jj = c * KCAP + lax.broadcasted_iota(i32, (T, KCAP), 1)
            gt = srow_c > scol
            tie = (srow_c == scol) & (jj < ii)
            rank = rank + jnp.sum((gt | tie).astype(f32), axis=1, keepdims=True)
        p_iota = lax.broadcasted_iota(i32, (T, KCAP), 1).astype(f32)
        oh = rank == p_iota
        fli = (b * T + lax.broadcasted_iota(i32, (T, KCAP), 0)).astype(f32)
        idx_row = jnp.sum(jnp.where(oh, fli, 0.0), axis=0, keepdims=True)
        sc_row = jnp.sum(jnp.where(oh, scol, 0.0), axis=0, keepdims=True)
        idx_i = idx_row.astype(i32)
        idx_ref[b:b + 1, :] = idx_i
        idx2_ref[b:b + 1, :] = idx_i >> 1
        par_ref[b:b + 1, :] = (idx_i & 1).astype(f32)
        gate_ref[b:b + 1, :] = 1.0 / (1.0 + jnp.exp(-sc_row))


_NW = 32
_GW = N // _NW
_SLAB = BT // _NW
_CCH = 64


@functools.cache
def _sc_kernels():
    vmesh = plsc.VectorSubcoreMesh(core_axis_name="c", subcore_axis_name="s")

    @functools.partial(
        pl.kernel,
        out_type=(
            jax.ShapeDtypeStruct((N, D), f32),
            jax.ShapeDtypeStruct((N, 2 * DH), f32),
            jax.ShapeDtypeStruct((N, 2 * DH), f32),
        ),
        mesh=vmesh,
        scratch_types=[
            pltpu.VMEM((_GW,), i32),
            pltpu.VMEM((_GW,), i32),
            pltpu.VMEM((_GW, D), f32),
            pltpu.VMEM((_GW, 2 * DH), f32),
            pltpu.VMEM((_GW, 2 * DH), f32),
            pltpu.SemaphoreType.DMA,
        ],
    )
    def sc_gather(hid_hbm, cp_hbm, sp_hbm, idx_hbm, idx2_hbm,
                  sel_hbm, cpg_hbm, spg_hbm,
                  idx_v, idx2_v, rows_v, c_v, s_v, sem):
        wid = lax.axis_index("s") * 2 + lax.axis_index("c")
        base = wid * _GW
        pltpu.sync_copy(idx_hbm.at[pl.ds(base, _GW)], idx_v)
        pltpu.sync_copy(idx2_hbm.at[pl.ds(base, _GW)], idx2_v)
        pltpu.async_copy(hid_hbm.at[idx_v], rows_v, sem).wait()
        pltpu.async_copy(cp_hbm.at[idx2_v], c_v, sem).wait()
        pltpu.async_copy(sp_hbm.at[idx2_v], s_v, sem).wait()
        pltpu.sync_copy(rows_v, sel_hbm.at[pl.ds(base, _GW)])
        pltpu.sync_copy(c_v, cpg_hbm.at[pl.ds(base, _GW)])
        pltpu.sync_copy(s_v, spg_hbm.at[pl.ds(base, _GW)])

    @functools.partial(
        pl.kernel,
        out_type=jax.ShapeDtypeStruct((BT, D), f32),
        mesh=vmesh,
        scratch_types=[
            pltpu.VMEM((_CCH, D), f32),
            pltpu.VMEM((_CCH, D), f32),
            pltpu.SemaphoreType.DMA,
            pltpu.SemaphoreType.DMA,
            pltpu.SemaphoreType.DMA,
            pltpu.SemaphoreType.DMA,
        ],
    )
    def sc_copy(hid_hbm, out_hbm, buf_a, buf_b, lsem_a, lsem_b, ssem_a, ssem_b):
        c = lax.axis_index("c")
        s = lax.axis_index("s")
        slab = (c * 16 + s) * _SLAB
        bufs = (buf_a, buf_b)
        lsems = (lsem_a, lsem_b)
        ssems = (ssem_a, ssem_b)
        prev_store = [None, None]
        for i in range(_SLAB // _CCH):
            j = i % 2
            if prev_store[j] is not None:
                prev_store[j].wait()
            pltpu.async_copy(hid_hbm.at[pl.ds(slab + i * _CCH, _CCH)],
                             bufs[j], lsems[j]).wait()
            prev_store[j] = pltpu.async_copy(
                bufs[j], out_hbm.at[pl.ds(slab + i * _CCH, _CCH)], ssems[j])
        for j in range(2):
            if prev_store[j] is not None:
                prev_store[j].wait()

    def _scatter_body(init_ref, upd_ref, idx_ref, out_ref, idx_v, buf, sem):
        del init_ref
        c = lax.axis_index("c")
        s = lax.axis_index("s")
        base = (c * 16 + s) * _GW
        pltpu.sync_copy(idx_ref.at[pl.ds(base, _GW)], idx_v)
        pltpu.async_copy(upd_ref.at[pl.ds(base, _GW)], buf, sem).wait()
        pltpu.sync_copy(buf, out_ref.at[idx_v])

    sc_scatter_inplace = _mpmd._mpmd_map(
        [(vmesh, _scatter_body)],
        jax.ShapeDtypeStruct((BT, D), f32),
        input_output_aliases={0: 0},
        scratch_types=[
            pltpu.VMEM((_GW,), i32),
            pltpu.VMEM((_GW, D), f32),
            pltpu.SemaphoreType.DMA,
        ],
    )

    def sc_scatter(hid2, upd, selidx):
        out_init = sc_copy(hid2)
        return sc_scatter_inplace(out_init, upd, selidx)

    return sc_gather, sc_scatter


def _rotate_half_full(x):
    left = pltpu.roll(x, x.shape[1] - 32, 1)
    right = pltpu.roll(x, 32, 1)
    lane = lax.broadcasted_iota(i32, x.shape, 1)
    return jnp.where((lane & 32) == 0, -left, right)


def _qkv_body(sel_ref, ln1_ref, wq_ref, bq_ref, wk_ref, bk_ref,
              wv_ref, bv_ref, cp_ref, sp_ref, par_ref,
              q_ref, k_ref, v_ref):
    x = sel_ref[...]
    ms = jnp.mean(x * x, axis=1, keepdims=True)
    hn = (x * lax.rsqrt(ms + EPS) * ln1_ref[...]).astype(bf16)
    q = jnp.dot(hn, wq_ref[...], preferred_element_type=f32) + bq_ref[...]
    k = jnp.dot(hn, wk_ref[...], preferred_element_type=f32) + bk_ref[...]
    v = jnp.dot(hn, wv_ref[...], preferred_element_type=f32) + bv_ref[...]
    par = par_ref[...]
    cp = cp_ref[...]
    sp = sp_ref[...]
    cos = jnp.where(par > 0.5, cp[:, DH:], cp[:, :DH])
    sin = jnp.where(par > 0.5, sp[:, DH:], sp[:, :DH])
    cosf = jnp.concatenate([cos] * H, axis=1)
    sinf = jnp.concatenate([sin] * H, axis=1)
    qr = (q * cosf + _rotate_half_full(q) * sinf) * 0.125
    kr = k * cosf + _rotate_half_full(k) * sinf
    q_ref[...] = qr.astype(bf16)
    k_ref[...] = kr.astype(bf16)
    v_ref[...] = v.astype(bf16)


def _attn_body(q_ref, k_ref, v_ref, o_ref):
    outs = []
    for hh in range(2):
        q1 = q_ref[:, hh * DH:(hh + 1) * DH]
        k1 = k_ref[:, hh * DH:(hh + 1) * DH]
        v1 = v_ref[:, hh * DH:(hh + 1) * DH]
        logits = lax.dot_general(q1, k1, (((1,), (1,)), ((), ())),
                                 preferred_element_type=f32)
        m = jnp.max(logits, axis=1, keepdims=True)
        e = jnp.exp(logits - m).astype(bf16)
        s = jnp.sum(e, axis=1, keepdims=True).astype(f32)
        oraw = jnp.dot(e, v1, preferred_element_type=f32)
        outs.append(oraw / s)
    o_ref[...] = jnp.concatenate(outs, axis=1).astype(bf16)


def _oproj_body(o_ref, wo_ref, sel_ref, ln2_ref, gate_ref, p_ref, hn2_ref):
    sl = sel_ref[...]
    h1 = jnp.dot(o_ref[...], wo_ref[...], preferred_element_type=f32) + sl
    p_ref[...] = sl + (h1 - sl) * gate_ref[...]
    ms = jnp.mean(h1 * h1, axis=1, keepdims=True)
    hn2_ref[...] = (h1 * lax.rsqrt(ms + EPS) * ln2_ref[...]).astype(bf16)


def _mlp_body(hn2_ref, wg_ref, wu_ref, wd_ref, p_ref, gate_ref, out_ref):
    fidx = pl.program_id(1)
    hn2 = hn2_ref[...]
    g = jnp.dot(hn2, wg_ref[...], preferred_element_type=f32)
    u = jnp.dot(hn2, wu_ref[...], preferred_element_type=f32)
    a = (g * (1.0 / (1.0 + jnp.exp(-g))) * u).astype(bf16)
    part = jnp.dot(a, wd_ref[...], preferred_element_type=f32)

    @pl.when(fidx == 0)
    def _():
        out_ref[...] = part

    @pl.when(fidx == 1)
    def _():
        out_ref[...] = p_ref[...] + (out_ref[...] + part) * gate_ref[...]


def kernel(hidden_states, original, posterior, prior, cos, sin,
           beta_ce, beta_cu, ce_offset, ln1_w, ln2_w,
           Wq, bq, Wk, bk, Wv, bv, Wo, Wg, Wu, Wd):
    hid2 = hidden_states.reshape(BT, D)
    cpair = cos.reshape(BT // 2, 2 * DH)
    spair = sin.reshape(BT // 2, 2 * DH)
    betas = jnp.stack([beta_cu, beta_ce, ce_offset]).reshape(1, 3)

    rchunk = 2048
    scores_col = pl.pallas_call(
        _scores_body,
        grid=(BT // rchunk,),
        in_specs=[
            pl.BlockSpec(memory_space=pltpu.SMEM),
            pl.BlockSpec((rchunk, D), lambda i: (i, 0)),
            pl.BlockSpec((rchunk, D), lambda i: (i, 0)),
            pl.BlockSpec((rchunk, D), lambda i: (i, 0)),
        ],
        out_specs=pl.BlockSpec((rchunk, 1), lambda i: (i, 0)),
        out_shape=jax.ShapeDtypeStruct((BT, 1), f32),
    )(betas, original.reshape(BT, D), posterior.reshape(BT, D),
      prior.reshape(BT, D))

    selidx4, selidx24, par4, gates4 = pl.pallas_call(
        _topk_body,
        in_specs=[pl.BlockSpec((BT, 1), lambda: (0, 0)),
                  pl.BlockSpec((1, BT), lambda: (0, 0))],
        out_specs=[pl.BlockSpec((B, KCAP), lambda: (0, 0))] * 4,
        out_shape=(jax.ShapeDtypeStruct((B, KCAP), i32),
                   jax.ShapeDtypeStruct((B, KCAP), i32),
                   jax.ShapeDtypeStruct((B, KCAP), f32),
                   jax.ShapeDtypeStruct((B, KCAP), f32)),
    )(scores_col, scores_col.reshape(1, BT))
    selidx = selidx4.reshape(N)
    selidx2 = selidx24.reshape(N)
    parity = par4.reshape(N, 1)
    gates = gates4.reshape(N, 1)

    sc_gather, sc_scatter = _sc_kernels()
    sel, cpg, spg = sc_gather(hid2, cpair, spair, selidx, selidx2)

    qrows = N // 2
    q2, k2, v2 = pl.pallas_call(
        _qkv_body,
        grid=(2,),
        in_specs=[
            pl.BlockSpec((qrows, D), lambda r: (r, 0)),
            pl.BlockSpec((1, D), lambda r: (0, 0)),
            pl.BlockSpec((D, D), lambda r: (0, 0)),
            pl.BlockSpec((1, D), lambda r: (0, 0)),
            pl.BlockSpec((D, D), lambda r: (0, 0)),
            pl.BlockSpec((1, D), lambda r: (0, 0)),
            pl.BlockSpec((D, D), lambda r: (0, 0)),
            pl.BlockSpec((1, D), lambda r: (0, 0)),
            pl.BlockSpec((qrows, 2 * DH), lambda r: (r, 0)),
            pl.BlockSpec((qrows, 2 * DH), lambda r: (r, 0)),
            pl.BlockSpec((qrows, 1), lambda r: (r, 0)),
        ],
        out_specs=[pl.BlockSpec((qrows, D), lambda r: (r, 0))] * 3,
        out_shape=(jax.ShapeDtypeStruct((N, D), bf16),) * 3,
    )(sel, ln1_w.reshape(1, D), Wq.astype(bf16), bq.reshape(1, D),
      Wk.astype(bf16), bk.reshape(1, D), Wv.astype(bf16), bv.reshape(1, D),
      cpg, spg, parity)

    o2 = pl.pallas_call(
        _attn_body,
        grid=(H // 2,),
        in_specs=[pl.BlockSpec((N, 2 * DH), lambda h: (0, h))] * 3,
        out_specs=pl.BlockSpec((N, 2 * DH), lambda h: (0, h)),
        out_shape=jax.ShapeDtypeStruct((N, D), bf16),
    )(q2, k2, v2)

    orows = N // 2
    pgated, hn2 = pl.pallas_call(
        _oproj_body,
        grid=(2,),
        in_specs=[
            pl.BlockSpec((orows, D), lambda r: (r, 0)),
            pl.BlockSpec((D, D), lambda r: (0, 0)),
            pl.BlockSpec((orows, D), lambda r: (r, 0)),
            pl.BlockSpec((1, D), lambda r: (0, 0)),
            pl.BlockSpec((orows, 1), lambda r: (r, 0)),
        ],
        out_specs=[pl.BlockSpec((orows, D), lambda r: (r, 0))] * 2,
        out_shape=(jax.ShapeDtypeStruct((N, D), f32),
                   jax.ShapeDtypeStruct((N, D), bf16)),
    )(o2, Wo.astype(bf16), sel, ln2_w.reshape(1, D), gates)

    mrows = N // 2
    fchunk = FF // 2
    upd = pl.pallas_call(
        _mlp_body,
        grid=(2, 2),
        in_specs=[
            pl.BlockSpec((mrows, D), lambda r, fc: (r, 0)),
            pl.BlockSpec((D, fchunk), lambda r, fc: (0, fc)),
            pl.BlockSpec((D, fchunk), lambda r, fc: (0, fc)),
            pl.BlockSpec((fchunk, D), lambda r, fc: (fc, 0)),
            pl.BlockSpec((mrows, D), lambda r, fc: (r, 0)),
            pl.BlockSpec((mrows, 1), lambda r, fc: (r, 0)),
        ],
        out_specs=pl.BlockSpec((mrows, D), lambda r, fc: (r, 0)),
        out_shape=jax.ShapeDtypeStruct((N, D), f32),
    )(hn2, Wg.astype(bf16), Wu.astype(bf16), Wd.astype(bf16), pgated, gates)

    out2 = sc_scatter(hid2, upd, selidx)
    return out2.reshape(B, T, D)
